# Initial kernel scaffold; baseline (speedup 1.0000x reference)
import functools

import jax
import jax.numpy as jnp
from jax import lax
from jax.experimental import pallas as pl
from jax.experimental.pallas import tpu as pltpu
from jax.experimental.pallas import tpu_sc as plsc

N = 512
E = 8192
H = 16
D = 256
NEG = jnp.float32(-200000000.0)

_MESH = plsc.VectorSubcoreMesh(core_axis_name="c", subcore_axis_name="s")


def _adj_body(ei_hbm, adj_hbm, ei_v, slab_v):
    wid = lax.axis_index("s") * 2 + lax.axis_index("c")
    lo = wid * 16
    pltpu.sync_copy(ei_hbm, ei_v)
    zeros = jnp.zeros((16,), jnp.int32)
    for r in range(16):
        def zbody(cc, _, r=r):
            slab_v[r, pl.ds(cc * 16, 16)] = zeros
            return 0
        lax.fori_loop(0, N // 16, zbody, 0)
    ones = jnp.ones((16,), jnp.int32)

    def ebody(e, _):
        s = ei_v[0, pl.ds(e * 16, 16)]
        d = ei_v[1, pl.ds(e * 16, 16)]
        m = (s >= lo) & (s < lo + 16)
        plsc.store_scatter(slab_v, [s - lo, d], ones, mask=m)
        return 0

    lax.fori_loop(0, E // 16, ebody, 0)
    pltpu.sync_copy(slab_v, adj_hbm.at[pl.ds(lo, 16), :])


_adj_call = functools.partial(
    pl.kernel,
    out_type=jax.ShapeDtypeStruct((N, N), jnp.int32),
    mesh=_MESH,
    scratch_types=[
        pltpu.VMEM((2, E), jnp.int32),
        pltpu.VMEM((16, N), jnp.int32),
    ],
)(_adj_body)


def _tc_body(adj_ref, x_ref, it_ref, ot_ref, rpt_ref, vd_ref,
             rel_ref, nf_ref, aug_ref, reach_ref, adjb_ref):
    adj = adj_ref[...]
    adjb_ref[...] = adj.astype(jnp.bfloat16)
    row = lax.broadcasted_iota(jnp.int32, (N, N), 0)
    col = lax.broadcasted_iota(jnp.int32, (N, N), 1)
    reach_ref[...] = (row == col).astype(jnp.bfloat16)
    rel_ref[...] = jnp.zeros((N, N), jnp.int32)

    def cond(c):
        return c[1] & (c[0] < N)

    def body(c):
        t, _ = c
        reach = reach_ref[...]
        rel_ref[...] = rel_ref[...] + (reach == 0).astype(jnp.int32)
        prod = jnp.dot(reach, adjb_ref[...],
                       preferred_element_type=jnp.float32)
        new = jnp.where((prod > 0.0) | (reach > 0),
                        jnp.bfloat16(1), jnp.bfloat16(0))
        reach_ref[...] = new
        return (t + 1, jnp.any(new != reach))

    lax.while_loop(cond, body, (jnp.int32(0), jnp.bool_(True)))

    reach = reach_ref[...]
    rel_ref[...] = jnp.where(reach > 0,
                             jnp.minimum(rel_ref[...], 510), 510)

    onesc = jnp.ones((N, 1), jnp.bfloat16)
    adjb = adjb_ref[...]
    in_deg = lax.dot_general(adjb, onesc, (((1,), (0,)), ((), ())),
                             preferred_element_type=jnp.float32)
    out_deg = lax.dot_general(adjb, onesc, (((0,), (0,)), ((), ())),
                              preferred_element_type=jnp.float32)
    in_deg = jnp.minimum(in_deg, 511.0).astype(jnp.int32)
    out_deg = jnp.minimum(out_deg, 511.0).astype(jnp.int32)
    oh_in = (in_deg == col).astype(jnp.float32)
    oh_out = (out_deg == col).astype(jnp.float32)
    nf_ref[...] = (x_ref[...]
                   + jnp.dot(oh_in, it_ref[...],
                             preferred_element_type=jnp.float32)
                   + jnp.dot(oh_out, ot_ref[...],
                             preferred_element_type=jnp.float32))

    v = lax.broadcasted_iota(jnp.int32, (N, H), 0)
    pen = jnp.where(v >= 20, NEG, jnp.float32(0.0))
    aug_ref[...] = rpt_ref[...] + vd_ref[...] + pen


def _tc_call(adj, x, it, ot, rpt, vd):
    return pl.pallas_call(
        _tc_body,
        out_shape=(
            jax.ShapeDtypeStruct((N, N), jnp.int32),
            jax.ShapeDtypeStruct((N, D), jnp.float32),
            jax.ShapeDtypeStruct((N, H), jnp.float32),
        ),
        scratch_shapes=[
            pltpu.VMEM((N, N), jnp.bfloat16),
            pltpu.VMEM((N, N), jnp.bfloat16),
        ],
    )(adj, x, it, ot, rpt, vd)


def _gab_body(rel_hbm, aug_hbm, out_hbm, aug_v, rp_v, buf_v):
    wid = lax.axis_index("s") * 2 + lax.axis_index("c")
    base = wid * 16
    pltpu.sync_copy(aug_hbm, aug_v)
    for ii in range(16):
        i = base + ii
        pltpu.sync_copy(rel_hbm.at[i], rp_v)

        def jbody(jv, _):
            idx = rp_v[pl.ds(jv * 16, 16)]
            for h in range(16):
                hv = jnp.full((16,), h, jnp.int32)
                buf_v[h, pl.ds(jv * 16, 16)] = plsc.load_gather(
                    aug_v, [idx, hv])
            return 0

        lax.fori_loop(0, N // 16, jbody, 0)
        pltpu.sync_copy(buf_v, out_hbm.at[i])


_gab_call = functools.partial(
    pl.kernel,
    out_type=jax.ShapeDtypeStruct((N, H, N), jnp.float32),
    mesh=_MESH,
    scratch_types=[
        pltpu.VMEM((N, H), jnp.float32),
        pltpu.VMEM((N,), jnp.int32),
        pltpu.VMEM((H, N), jnp.float32),
    ],
)(_gab_body)


def kernel(x, edge_feature, edge_index, in_deg_table, out_deg_table,
           rel_pos_table, virt_dist):
    del edge_feature
    adj = _adj_call(edge_index.astype(jnp.int32))
    rel_pos, node_feature, aug = _tc_call(
        adj, x, in_deg_table, out_deg_table, rel_pos_table, virt_dist)
    inner = _gab_call(rel_pos, aug)
    gab = jnp.pad(inner, ((1, 0), (0, 0), (1, 0)))
    return node_feature, gab


# speedup vs baseline: 9.6046x; 9.6046x over previous
import functools

import jax
import jax.numpy as jnp
from jax import lax
from jax.experimental import pallas as pl
from jax.experimental.pallas import tpu as pltpu
from jax.experimental.pallas import tpu_sc as plsc

N = 512
E = 8192
H = 16
D = 256
NEG = -200000000.0

_MESH = plsc.VectorSubcoreMesh(core_axis_name="c", subcore_axis_name="s")


def _adj_body(ei_hbm, adj_hbm, ei_v, slab_v):
    wid = lax.axis_index("s") * 2 + lax.axis_index("c")
    lo = wid * 16
    pltpu.sync_copy(ei_hbm, ei_v)
    zeros = jnp.zeros((16,), jnp.int32)
    for r in range(16):
        def zbody(cc, _, r=r):
            slab_v[r, pl.ds(cc * 16, 16)] = zeros
            return 0
        lax.fori_loop(0, N // 16, zbody, 0)
    ones = jnp.ones((16,), jnp.int32)

    def ebody(e, _):
        s = ei_v[0, pl.ds(e * 16, 16)]
        d = ei_v[1, pl.ds(e * 16, 16)]
        m = (s >= lo) & (s < lo + 16)
        plsc.store_scatter(slab_v, [s - lo, d], ones, mask=m)
        return 0

    lax.fori_loop(0, E // 16, ebody, 0)
    pltpu.sync_copy(slab_v, adj_hbm.at[pl.ds(lo, 16), :])


_adj_call = functools.partial(
    pl.kernel,
    out_type=jax.ShapeDtypeStruct((N, N), jnp.int32),
    mesh=_MESH,
    compiler_params=pltpu.CompilerParams(use_tc_tiling_on_sc=False, needs_layout_passes=False),
    scratch_types=[
        pltpu.VMEM((2, E), jnp.int32),
        pltpu.VMEM((16, N), jnp.int32),
    ],
)(_adj_body)


def _tc_body(adj_ref, x_ref, it_ref, ot_ref, rpt_ref, vd_ref,
             rel_ref, nf_ref, aug_ref, reach_ref, adjb_ref):
    adj = adj_ref[...]
    adjb_ref[...] = adj.astype(jnp.bfloat16)
    row = lax.broadcasted_iota(jnp.int32, (N, N), 0)
    col = lax.broadcasted_iota(jnp.int32, (N, N), 1)
    reach_ref[...] = (row == col).astype(jnp.float32)
    rel_ref[...] = jnp.zeros((N, N), jnp.int32)

    def cond(c):
        return c[1] & (c[0] < N)

    def body(c):
        t, _ = c
        reach = reach_ref[...]
        rel_ref[...] = rel_ref[...] + (reach == 0).astype(jnp.int32)
        prod = jnp.dot(reach.astype(jnp.bfloat16), adjb_ref[...],
                       preferred_element_type=jnp.float32)
        new = jnp.where(prod + reach > 0.0, jnp.float32(1), jnp.float32(0))
        reach_ref[...] = new
        return (t + 1, jnp.any(new != reach))

    lax.while_loop(cond, body, (jnp.int32(0), jnp.bool_(True)))

    reach = reach_ref[...]
    rel_ref[...] = jnp.where(reach > 0,
                             jnp.minimum(rel_ref[...], 510), 510)

    onesc = jnp.ones((N, 1), jnp.bfloat16)
    adjb = adjb_ref[...]
    in_deg = lax.dot_general(adjb, onesc, (((1,), (0,)), ((), ())),
                             preferred_element_type=jnp.float32)
    out_deg = lax.dot_general(adjb, onesc, (((0,), (0,)), ((), ())),
                              preferred_element_type=jnp.float32)
    in_deg = jnp.minimum(in_deg, 511.0).astype(jnp.int32)
    out_deg = jnp.minimum(out_deg, 511.0).astype(jnp.int32)
    oh_in = (in_deg == col).astype(jnp.float32)
    oh_out = (out_deg == col).astype(jnp.float32)
    nf_ref[...] = (x_ref[...]
                   + jnp.dot(oh_in, it_ref[...],
                             preferred_element_type=jnp.float32)
                   + jnp.dot(oh_out, ot_ref[...],
                             preferred_element_type=jnp.float32))

    v = lax.broadcasted_iota(jnp.int32, (N, H), 0)
    pen = jnp.where(v >= 20, jnp.float32(NEG), jnp.float32(0.0))
    aug_ref[...] = rpt_ref[...] + vd_ref[...] + pen


def _tc_call(adj, x, it, ot, rpt, vd):
    return pl.pallas_call(
        _tc_body,
        out_shape=(
            jax.ShapeDtypeStruct((N, N), jnp.int32),
            jax.ShapeDtypeStruct((N, D), jnp.float32),
            jax.ShapeDtypeStruct((N, H), jnp.float32),
        ),
        scratch_shapes=[
            pltpu.VMEM((N, N), jnp.float32),
            pltpu.VMEM((N, N), jnp.bfloat16),
        ],
    )(adj, x, it, ot, rpt, vd)


def _gab_body(rel_hbm, aug_hbm, out_hbm, aug_v, rp_v, buf_v):
    wid = lax.axis_index("s") * 2 + lax.axis_index("c")
    base = wid * 16
    pltpu.sync_copy(aug_hbm, aug_v)
    for ii in range(16):
        i = base + ii
        pltpu.sync_copy(rel_hbm.at[i], rp_v)

        def jbody(jv, _):
            idx = rp_v[pl.ds(jv * 16, 16)]
            for h in range(16):
                hv = jnp.full((16,), h, jnp.int32)
                buf_v[h, pl.ds(jv * 16, 16)] = plsc.load_gather(
                    aug_v, [idx, hv])
            return 0

        lax.fori_loop(0, N // 16, jbody, 0)
        pltpu.sync_copy(buf_v, out_hbm.at[i])


_gab_call = functools.partial(
    pl.kernel,
    out_type=jax.ShapeDtypeStruct((N, H, N), jnp.float32),
    mesh=_MESH,
    compiler_params=pltpu.CompilerParams(use_tc_tiling_on_sc=False, needs_layout_passes=False),
    scratch_types=[
        pltpu.VMEM((N, H), jnp.float32),
        pltpu.VMEM((N,), jnp.int32),
        pltpu.VMEM((H, N), jnp.float32),
    ],
)(_gab_body)


def kernel(x, edge_feature, edge_index, in_deg_table, out_deg_table,
           rel_pos_table, virt_dist):
    del edge_feature
    adj = _adj_call(edge_index.astype(jnp.int32))
    rel_pos, node_feature, aug = _tc_call(
        adj, x, in_deg_table, out_deg_table, rel_pos_table, virt_dist)
    inner = _gab_call(rel_pos, aug)
    gab = jnp.pad(inner, ((1, 0), (0, 0), (1, 0)))
    return node_feature, gab


# speedup vs baseline: 14.7673x; 1.5375x over previous
import functools

import jax
import jax.numpy as jnp
from jax import lax
from jax.experimental import pallas as pl
from jax.experimental.pallas import tpu as pltpu
from jax.experimental.pallas import tpu_sc as plsc

N = 512
E = 8192
H = 16
D = 256
NEG = -200000000.0

_MESH = plsc.VectorSubcoreMesh(core_axis_name="c", subcore_axis_name="s")


def _adj_body(ei_hbm, adj_hbm, ei_v, slab_v):
    wid = lax.axis_index("s") * 2 + lax.axis_index("c")
    lo = wid * 16
    pltpu.sync_copy(ei_hbm, ei_v)
    zeros = jnp.zeros((16,), jnp.int32)
    for r in range(16):
        def zbody(cc, _, r=r):
            slab_v[r, pl.ds(cc * 16, 16)] = zeros
            return 0
        lax.fori_loop(0, N // 16, zbody, 0)
    ones = jnp.ones((16,), jnp.int32)

    def ebody(e, _):
        s = ei_v[0, pl.ds(e * 16, 16)]
        d = ei_v[1, pl.ds(e * 16, 16)]
        m = (s >= lo) & (s < lo + 16)
        plsc.store_scatter(slab_v, [s - lo, d], ones, mask=m)
        return 0

    lax.fori_loop(0, E // 16, ebody, 0)
    pltpu.sync_copy(slab_v, adj_hbm.at[pl.ds(lo, 16), :])


_adj_call = functools.partial(
    pl.kernel,
    out_type=jax.ShapeDtypeStruct((N, N), jnp.int32),
    mesh=_MESH,
    compiler_params=pltpu.CompilerParams(use_tc_tiling_on_sc=False, needs_layout_passes=False),
    scratch_types=[
        pltpu.VMEM((2, E), jnp.int32),
        pltpu.VMEM((16, N), jnp.int32),
    ],
)(_adj_body)


def _tc_body(adj_ref, x_ref, it_ref, ot_ref, rpt_ref, vd_ref,
             rel_ref, nf_ref, aug_ref, reach_ref, adjb_ref):
    adj = adj_ref[...]
    adjb_ref[...] = adj.astype(jnp.bfloat16)
    row = lax.broadcasted_iota(jnp.int32, (N, N), 0)
    col = lax.broadcasted_iota(jnp.int32, (N, N), 1)
    reach_ref[...] = (row == col).astype(jnp.float32)
    rel_ref[...] = jnp.zeros((N, N), jnp.int32)

    def cond(c):
        return c[1] & (c[0] < N)

    def body(c):
        t, _ = c
        reach = reach_ref[...]
        rel_ref[...] = rel_ref[...] + (reach == 0).astype(jnp.int32)
        prod = jnp.dot(reach.astype(jnp.bfloat16), adjb_ref[...],
                       preferred_element_type=jnp.float32)
        new = jnp.where(prod + reach > 0.0, jnp.float32(1), jnp.float32(0))
        reach_ref[...] = new
        return (t + 1, jnp.any(new != reach))

    lax.while_loop(cond, body, (jnp.int32(0), jnp.bool_(True)))

    reach = reach_ref[...]
    rel_ref[...] = jnp.where(reach > 0,
                             jnp.minimum(rel_ref[...], 510), 510)

    onesc = jnp.ones((N, 1), jnp.bfloat16)
    adjb = adjb_ref[...]
    in_deg = lax.dot_general(adjb, onesc, (((1,), (0,)), ((), ())),
                             preferred_element_type=jnp.float32)
    out_deg = lax.dot_general(adjb, onesc, (((0,), (0,)), ((), ())),
                              preferred_element_type=jnp.float32)
    in_deg = jnp.minimum(in_deg, 511.0).astype(jnp.int32)
    out_deg = jnp.minimum(out_deg, 511.0).astype(jnp.int32)
    oh_in = (in_deg == col).astype(jnp.float32)
    oh_out = (out_deg == col).astype(jnp.float32)
    nf_ref[...] = (x_ref[...]
                   + jnp.dot(oh_in, it_ref[...],
                             preferred_element_type=jnp.float32)
                   + jnp.dot(oh_out, ot_ref[...],
                             preferred_element_type=jnp.float32))

    v = lax.broadcasted_iota(jnp.int32, (N, H), 0)
    pen = jnp.where(v >= 20, jnp.float32(NEG), jnp.float32(0.0))
    aug_ref[...] = rpt_ref[...] + vd_ref[...] + pen


def _tc_call(adj, x, it, ot, rpt, vd):
    return pl.pallas_call(
        _tc_body,
        out_shape=(
            jax.ShapeDtypeStruct((N, N), jnp.int32),
            jax.ShapeDtypeStruct((N, D), jnp.float32),
            jax.ShapeDtypeStruct((N, H), jnp.float32),
        ),
        scratch_shapes=[
            pltpu.VMEM((N, N), jnp.float32),
            pltpu.VMEM((N, N), jnp.bfloat16),
        ],
    )(adj, x, it, ot, rpt, vd)


def _gab_body(rel_hbm, aug_hbm, out_hbm, aug_v, rp_v, buf0, buf1, sem0,
              sem1):
    wid = lax.axis_index("s") * 2 + lax.axis_index("c")
    base = wid * 16
    pltpu.sync_copy(aug_hbm, aug_v)
    pltpu.sync_copy(rel_hbm.at[pl.ds(base, 16), :], rp_v)

    zeros = jnp.zeros((16,), jnp.float32)
    for h in range(16):
        def zb(c, _, h=h):
            buf0[h, pl.ds(c * 16, 16)] = zeros
            return 0
        lax.fori_loop(0, N // 16, zb, 0)
        buf0[h, pl.ds(497, 16)] = zeros
        buf1[h, pl.ds(0, 16)] = zeros

    @pl.when(wid == 0)
    def _():
        pltpu.sync_copy(buf0, out_hbm.at[0])

    bufs = (buf0, buf1)
    sems = (sem0, sem1)
    copies = [None, None]
    for ii in range(16):
        buf = bufs[ii % 2]
        if copies[ii % 2] is not None:
            copies[ii % 2].wait()

        @plsc.parallel_loop(0, N // 16, unroll=2)
        def jbody(jv, buf=buf, ii=ii):
            idx16 = rp_v[ii, pl.ds(jv * 16, 16)] * 16
            for h in range(16):
                buf[h, pl.ds(jv * 16 + 1, 16)] = plsc.load_gather(
                    aug_v, [idx16 + h])

        copies[ii % 2] = pltpu.async_copy(
            buf, out_hbm.at[base + 1 + ii], sems[ii % 2])
    copies[0].wait()
    copies[1].wait()


_gab_call = functools.partial(
    pl.kernel,
    out_type=jax.ShapeDtypeStruct((N + 1, H, N + 1), jnp.float32),
    mesh=_MESH,
    compiler_params=pltpu.CompilerParams(use_tc_tiling_on_sc=False,
                                         needs_layout_passes=False),
    scratch_types=[
        pltpu.VMEM((N * H,), jnp.float32),
        pltpu.VMEM((16, N), jnp.int32),
        pltpu.VMEM((H, N + 1), jnp.float32),
        pltpu.VMEM((H, N + 1), jnp.float32),
        pltpu.SemaphoreType.DMA,
        pltpu.SemaphoreType.DMA,
    ],
)(_gab_body)


def kernel(x, edge_feature, edge_index, in_deg_table, out_deg_table,
           rel_pos_table, virt_dist):
    del edge_feature
    adj = _adj_call(edge_index.astype(jnp.int32))
    rel_pos, node_feature, aug = _tc_call(
        adj, x, in_deg_table, out_deg_table, rel_pos_table, virt_dist)
    gab = _gab_call(rel_pos, aug.reshape(-1))
    return node_feature, gab
